# Initial kernel scaffold; baseline (speedup 1.0000x reference)
import jax
import jax.numpy as jnp
from jax.experimental import pallas as pl


def kernel(q, k, v, w_q, b_q, w_k, b_k, w_v, b_v, w_o, b_o):
    raise NotImplementedError("write your pallas kernel here")


# speedup vs baseline: 1.5664x; 1.5664x over previous
import math

import jax
import jax.numpy as jnp
import numpy as np
from jax.experimental import pallas as pl
from jax.experimental.pallas import tpu as pltpu

B, LQ, LK = 8, 32, 8192
D_MODEL, N_HEAD, SF = 768, 12, 5
D_T = D_MODEL // N_HEAD
LOG_LK = int(np.ceil(np.log1p(LK)))
LOG_LQ = int(np.ceil(np.log1p(LQ)))
U_PART = min(SF * LQ * LOG_LK, LK)
U_Q = min(SF * LOG_LQ, LQ)
BH = B * N_HEAD
R = N_HEAD * LQ
BK = 2048
NKB = LK // BK


def _qproj_body(q_ref, wq_ref, bq_ref, out_ref):
    out_ref[...] = (
        jnp.dot(q_ref[...], wq_ref[...].T, preferred_element_type=jnp.float32)
        + bq_ref[...]
    )


def _sample_body(qf_ref, ks_ref, wk_ref, bk_ref, m_ref):
    ksp = (jnp.dot(ks_ref[0], wk_ref[...].T, preferred_element_type=jnp.float32)
           + bk_ref[...])
    for h in range(N_HEAD):
        ksp_h = ksp[:, h * D_T:(h + 1) * D_T]
        qk = jnp.dot(qf_ref[0, h], ksp_h.T, preferred_element_type=jnp.float32)
        m_ref[0, h] = jnp.max(qk, axis=-1) - jnp.sum(qk, axis=-1) / LK


def _attn_body(qf_ref, wk_ref, k_ref, v_ref, out_ref,
               gb_ref, acc_ref, mx_ref, sm_ref):
    kb = pl.program_id(1)

    @pl.when(kb == 0)
    def _init():
        for h in range(N_HEAD):
            gb_ref[h * LQ:(h + 1) * LQ, :] = jnp.dot(
                qf_ref[0, h], wk_ref[h * D_T:(h + 1) * D_T, :],
                preferred_element_type=jnp.float32).astype(jnp.bfloat16)
        acc_ref[...] = jnp.zeros_like(acc_ref)
        mx_ref[...] = jnp.full_like(mx_ref, -jnp.inf)
        sm_ref[...] = jnp.zeros_like(sm_ref)

    s = jax.lax.dot_general(
        gb_ref[...], k_ref[0].astype(jnp.bfloat16),
        (((1,), (1,)), ((), ())),
        preferred_element_type=jnp.float32) * (1.0 / math.sqrt(D_T))
    m_prev = mx_ref[...]
    m_new = jnp.maximum(m_prev, jnp.max(s, axis=-1, keepdims=True))
    alpha = jnp.exp(m_prev - m_new)
    p = jnp.exp(s - m_new)
    mx_ref[...] = m_new
    sm_ref[...] = sm_ref[...] * alpha + jnp.sum(p, axis=-1, keepdims=True)
    acc_ref[...] = acc_ref[...] * alpha + jax.lax.dot_general(
        p.astype(jnp.bfloat16), v_ref[0].astype(jnp.bfloat16),
        (((1,), (0,)), ((), ())), preferred_element_type=jnp.float32)

    @pl.when(kb == NKB - 1)
    def _fini():
        out_ref[0] = acc_ref[...] / sm_ref[...]


def _epilogue_body(a_ref, wv_ref, bv_ref, wo_ref, bo_ref, out_ref):
    cat = []
    for h in range(N_HEAD):
        cat.append(jnp.dot(
            a_ref[0, h], wv_ref[h * D_T:(h + 1) * D_T, :].T,
            preferred_element_type=jnp.float32))
    cat = jnp.concatenate(cat, axis=-1) + bv_ref[...]
    out_ref[0] = jnp.dot(cat, wo_ref[...].T,
                         preferred_element_type=jnp.float32) + bo_ref[...]


def kernel(q, k, v, w_q, b_q, w_k, b_k, w_v, b_v, w_o, b_o):
    qp = pl.pallas_call(
        _qproj_body,
        out_shape=jax.ShapeDtypeStruct((B * LQ, D_MODEL), jnp.float32),
    )(q.reshape(B * LQ, D_MODEL), w_q, b_q)
    qf = qp.reshape(B, N_HEAD, LQ, D_T)

    idx = jax.random.randint(jax.random.key(42), (U_PART,), 0, LK)
    idx = jnp.sort(idx)
    ks = jnp.take(k, idx, axis=1)
    m = pl.pallas_call(
        _sample_body,
        grid=(B,),
        in_specs=[
            pl.BlockSpec((1, N_HEAD, LQ, D_T), lambda b: (b, 0, 0, 0)),
            pl.BlockSpec((1, U_PART, D_MODEL), lambda b: (b, 0, 0)),
            pl.BlockSpec((D_MODEL, D_MODEL), lambda b: (0, 0)),
            pl.BlockSpec((D_MODEL,), lambda b: (0,)),
        ],
        out_specs=pl.BlockSpec((1, N_HEAD, LQ), lambda b: (b, 0, 0)),
        out_shape=jax.ShapeDtypeStruct((B, N_HEAD, LQ), jnp.float32),
    )(qf, ks, w_k, b_k)

    a_all = pl.pallas_call(
        _attn_body,
        grid=(B, NKB),
        in_specs=[
            pl.BlockSpec((1, N_HEAD, LQ, D_T), lambda b, kb: (b, 0, 0, 0)),
            pl.BlockSpec((D_MODEL, D_MODEL), lambda b, kb: (0, 0)),
            pl.BlockSpec((1, BK, D_MODEL), lambda b, kb: (b, kb, 0)),
            pl.BlockSpec((1, BK, D_MODEL), lambda b, kb: (b, kb, 0)),
        ],
        out_specs=pl.BlockSpec((1, R, D_MODEL), lambda b, kb: (b, 0, 0)),
        out_shape=jax.ShapeDtypeStruct((B, R, D_MODEL), jnp.float32),
        scratch_shapes=[
            pltpu.VMEM((R, D_MODEL), jnp.bfloat16),
            pltpu.VMEM((R, D_MODEL), jnp.float32),
            pltpu.VMEM((R, 1), jnp.float32),
            pltpu.VMEM((R, 1), jnp.float32),
        ],
        compiler_params=pltpu.CompilerParams(
            dimension_semantics=("parallel", "arbitrary")),
    )(qf, w_k, k, v)

    _, m_top = jax.lax.top_k(-m.reshape(BH, LQ), U_Q)
    a_sel = jnp.take_along_axis(
        a_all.reshape(BH, LQ, D_MODEL), m_top[:, :, None], axis=1)
    a_sel = a_sel.reshape(B, N_HEAD, U_Q, D_MODEL)

    out = pl.pallas_call(
        _epilogue_body,
        grid=(B,),
        in_specs=[
            pl.BlockSpec((1, N_HEAD, U_Q, D_MODEL), lambda b: (b, 0, 0, 0)),
            pl.BlockSpec((D_MODEL, D_MODEL), lambda b: (0, 0)),
            pl.BlockSpec((D_MODEL,), lambda b: (0,)),
            pl.BlockSpec((D_MODEL, D_MODEL), lambda b: (0, 0)),
            pl.BlockSpec((D_MODEL,), lambda b: (0,)),
        ],
        out_specs=pl.BlockSpec((1, U_Q, D_MODEL), lambda b: (b, 0, 0)),
        out_shape=jax.ShapeDtypeStruct((B, U_Q, D_MODEL), jnp.float32),
    )(a_sel, w_v, b_v, w_o, b_o)
    return out


# speedup vs baseline: 1.9328x; 1.2339x over previous
import functools
import math

import jax
import jax.numpy as jnp
import numpy as np
from jax.experimental import pallas as pl
from jax.experimental.pallas import tpu as pltpu
from jax.experimental.pallas import tpu_sc as plsc

B, LQ, LK = 8, 32, 8192
D_MODEL, N_HEAD, SF = 768, 12, 5
D_T = D_MODEL // N_HEAD
LOG_LK = int(np.ceil(np.log1p(LK)))
LOG_LQ = int(np.ceil(np.log1p(LQ)))
U_PART = min(SF * LQ * LOG_LK, LK)
U_Q = min(SF * LOG_LQ, LQ)
BH = B * N_HEAD
R = N_HEAD * LQ
BK = 2048
NKB = LK // BK


_SC_INFO = plsc.get_sparse_core_info()
_NW = _SC_INFO.num_cores * _SC_INFO.num_subcores
_GROWS = B * U_PART
_RPW = _GROWS // _NW
_CHUNK = 80
_NCH = _RPW // _CHUNK


def _sc_gather_body(kflat_ref, gidx_ref, out_ref, idx_v, rows_v, sem):
    wid = jax.lax.axis_index("s") * _SC_INFO.num_cores + jax.lax.axis_index("c")
    for c in range(_NCH):
        base = wid * _RPW + c * _CHUNK
        pltpu.sync_copy(gidx_ref.at[pl.ds(base, _CHUNK)], idx_v)
        pltpu.async_copy(kflat_ref.at[idx_v], rows_v, sem).wait()
        pltpu.sync_copy(rows_v, out_ref.at[pl.ds(base, _CHUNK)])


_sc_gather = functools.partial(
    pl.kernel,
    mesh=plsc.VectorSubcoreMesh(core_axis_name="c", subcore_axis_name="s"),
    out_type=jax.ShapeDtypeStruct((_GROWS, D_MODEL), jnp.float32),
    scratch_types=[
        pltpu.VMEM((_CHUNK,), jnp.int32),
        pltpu.VMEM((_CHUNK, D_MODEL), jnp.float32),
        pltpu.SemaphoreType.DMA,
    ],
)(_sc_gather_body)


def _qproj_body(q_ref, wq_ref, bq_ref, out_ref):
    out_ref[...] = (
        jnp.dot(q_ref[...], wq_ref[...].T, preferred_element_type=jnp.float32)
        + bq_ref[...]
    )


def _sample_body(qf_ref, ks_ref, wk_ref, bk_ref, m_ref):
    ksp = (jnp.dot(ks_ref[0], wk_ref[...].T, preferred_element_type=jnp.float32)
           + bk_ref[...])
    for h in range(N_HEAD):
        ksp_h = ksp[:, h * D_T:(h + 1) * D_T]
        qk = jnp.dot(qf_ref[0, h], ksp_h.T, preferred_element_type=jnp.float32)
        m_ref[0, h] = jnp.max(qk, axis=-1) - jnp.sum(qk, axis=-1) / LK


def _attn_body(qf_ref, wk_ref, k_ref, v_ref, out_ref,
               gb_ref, acc_ref, mx_ref, sm_ref):
    kb = pl.program_id(1)

    @pl.when(kb == 0)
    def _init():
        for h in range(N_HEAD):
            gb_ref[h * LQ:(h + 1) * LQ, :] = jnp.dot(
                qf_ref[0, h], wk_ref[h * D_T:(h + 1) * D_T, :],
                preferred_element_type=jnp.float32).astype(jnp.bfloat16)
        acc_ref[...] = jnp.zeros_like(acc_ref)
        mx_ref[...] = jnp.full_like(mx_ref, -jnp.inf)
        sm_ref[...] = jnp.zeros_like(sm_ref)

    s = jax.lax.dot_general(
        gb_ref[...], k_ref[0].astype(jnp.bfloat16),
        (((1,), (1,)), ((), ())),
        preferred_element_type=jnp.float32) * (1.0 / math.sqrt(D_T))
    m_prev = mx_ref[...]
    m_new = jnp.maximum(m_prev, jnp.max(s, axis=-1, keepdims=True))
    alpha = jnp.exp(m_prev - m_new)
    p = jnp.exp(s - m_new)
    mx_ref[...] = m_new
    sm_ref[...] = sm_ref[...] * alpha + jnp.sum(p, axis=-1, keepdims=True)
    acc_ref[...] = acc_ref[...] * alpha + jax.lax.dot_general(
        p.astype(jnp.bfloat16), v_ref[0].astype(jnp.bfloat16),
        (((1,), (0,)), ((), ())), preferred_element_type=jnp.float32)

    @pl.when(kb == NKB - 1)
    def _fini():
        out_ref[0] = acc_ref[...] / sm_ref[...]


def _epilogue_body(a_ref, wv_ref, bv_ref, wo_ref, bo_ref, out_ref):
    cat = []
    for h in range(N_HEAD):
        cat.append(jnp.dot(
            a_ref[0, h], wv_ref[h * D_T:(h + 1) * D_T, :].T,
            preferred_element_type=jnp.float32))
    cat = jnp.concatenate(cat, axis=-1) + bv_ref[...]
    out_ref[0] = jnp.dot(cat, wo_ref[...].T,
                         preferred_element_type=jnp.float32) + bo_ref[...]


def kernel(q, k, v, w_q, b_q, w_k, b_k, w_v, b_v, w_o, b_o):
    qp = pl.pallas_call(
        _qproj_body,
        out_shape=jax.ShapeDtypeStruct((B * LQ, D_MODEL), jnp.float32),
    )(q.reshape(B * LQ, D_MODEL), w_q, b_q)
    qf = qp.reshape(B, N_HEAD, LQ, D_T)

    idx = jax.random.randint(jax.random.key(42), (U_PART,), 0, LK)
    idx = jnp.sort(idx)
    gidx = (jnp.arange(B, dtype=jnp.int32)[:, None] * LK + idx[None, :]).reshape(-1)
    ks = _sc_gather(k.reshape(B * LK, D_MODEL), gidx)
    ks = ks.reshape(B, U_PART, D_MODEL)
    m = pl.pallas_call(
        _sample_body,
        grid=(B,),
        in_specs=[
            pl.BlockSpec((1, N_HEAD, LQ, D_T), lambda b: (b, 0, 0, 0)),
            pl.BlockSpec((1, U_PART, D_MODEL), lambda b: (b, 0, 0)),
            pl.BlockSpec((D_MODEL, D_MODEL), lambda b: (0, 0)),
            pl.BlockSpec((D_MODEL,), lambda b: (0,)),
        ],
        out_specs=pl.BlockSpec((1, N_HEAD, LQ), lambda b: (b, 0, 0)),
        out_shape=jax.ShapeDtypeStruct((B, N_HEAD, LQ), jnp.float32),
    )(qf, ks, w_k, b_k)

    a_all = pl.pallas_call(
        _attn_body,
        grid=(B, NKB),
        in_specs=[
            pl.BlockSpec((1, N_HEAD, LQ, D_T), lambda b, kb: (b, 0, 0, 0)),
            pl.BlockSpec((D_MODEL, D_MODEL), lambda b, kb: (0, 0)),
            pl.BlockSpec((1, BK, D_MODEL), lambda b, kb: (b, kb, 0)),
            pl.BlockSpec((1, BK, D_MODEL), lambda b, kb: (b, kb, 0)),
        ],
        out_specs=pl.BlockSpec((1, R, D_MODEL), lambda b, kb: (b, 0, 0)),
        out_shape=jax.ShapeDtypeStruct((B, R, D_MODEL), jnp.float32),
        scratch_shapes=[
            pltpu.VMEM((R, D_MODEL), jnp.bfloat16),
            pltpu.VMEM((R, D_MODEL), jnp.float32),
            pltpu.VMEM((R, 1), jnp.float32),
            pltpu.VMEM((R, 1), jnp.float32),
        ],
        compiler_params=pltpu.CompilerParams(
            dimension_semantics=("parallel", "arbitrary")),
    )(qf, w_k, k, v)

    _, m_top = jax.lax.top_k(-m.reshape(BH, LQ), U_Q)
    a_sel = jnp.take_along_axis(
        a_all.reshape(BH, LQ, D_MODEL), m_top[:, :, None], axis=1)
    a_sel = a_sel.reshape(B, N_HEAD, U_Q, D_MODEL)

    out = pl.pallas_call(
        _epilogue_body,
        grid=(B,),
        in_specs=[
            pl.BlockSpec((1, N_HEAD, U_Q, D_MODEL), lambda b: (b, 0, 0, 0)),
            pl.BlockSpec((D_MODEL, D_MODEL), lambda b: (0, 0)),
            pl.BlockSpec((D_MODEL,), lambda b: (0,)),
            pl.BlockSpec((D_MODEL, D_MODEL), lambda b: (0, 0)),
            pl.BlockSpec((D_MODEL,), lambda b: (0,)),
        ],
        out_specs=pl.BlockSpec((1, U_Q, D_MODEL), lambda b: (b, 0, 0)),
        out_shape=jax.ShapeDtypeStruct((B, U_Q, D_MODEL), jnp.float32),
    )(a_sel, w_v, b_v, w_o, b_o)
    return out
